# Initial kernel scaffold; baseline (speedup 1.0000x reference)
import jax
import jax.numpy as jnp
from jax.experimental import pallas as pl


def kernel(x, Wq, bq, Wk, bk, Wv, bv, Wo, bo):
    raise NotImplementedError("write your pallas kernel here")


# speedup vs baseline: 3.0908x; 3.0908x over previous
import functools
import math

import jax
import jax.numpy as jnp
from jax import lax
from jax.experimental import pallas as pl
from jax.experimental.pallas import tpu as pltpu

L = 2048
D = 1024
H = 16
DH = 64
BL = 256
PAD = 1024
OFFSETS = tuple(2 ** j for j in range(11))
SCALE = 1.0 / math.sqrt(DH)


def _proj_kernel(x_ref, wq_ref, wk_ref, wv_ref, bq_ref, bk_ref, bv_ref,
                 q_ref, k_ref, v_ref, sv_ref):
    xb = x_ref[...]
    f32 = jnp.float32
    q_ref[...] = jnp.dot(xb, wq_ref[...], preferred_element_type=f32) + bq_ref[...]
    k_ref[...] = jnp.dot(xb, wk_ref[...], preferred_element_type=f32) + bk_ref[...]
    vb = jnp.dot(xb, wv_ref[...], preferred_element_type=f32) + bv_ref[...]
    v_ref[...] = vb
    sv = jnp.sum(vb, axis=0, keepdims=True)

    @pl.when(pl.program_id(0) == 0)
    def _():
        sv_ref[...] = sv

    @pl.when(pl.program_id(0) != 0)
    def _():
        sv_ref[...] += sv


def _attn_kernel(q_ref, kp_ref, vp_ref, sv_ref, wo_ref, bo_ref, o_ref):
    f32 = jnp.float32
    i0 = pl.program_id(0) * BL
    q = q_ref[...]

    sel = (lax.broadcasted_iota(jnp.int32, (D, H), 0) // DH
           == lax.broadcasted_iota(jnp.int32, (D, H), 1)).astype(f32)
    selT = (lax.broadcasted_iota(jnp.int32, (H, D), 1) // DH
            == lax.broadcasted_iota(jnp.int32, (H, D), 0)).astype(f32)

    kd = kp_ref[pl.ds(i0 + PAD, BL), :]
    vd = vp_ref[pl.ds(i0 + PAD, BL), :]
    s = jnp.dot(q * kd, sel, preferred_element_type=f32) * SCALE
    w = jnp.exp(s) - 1.0
    z = w + float(L)
    acc = jnp.dot(w, selT, preferred_element_type=f32) * vd

    for d in OFFSETS:
        if d % 8 == 0:
            ks = kp_ref[pl.ds(i0 + PAD - d, BL), :]
            vs = vp_ref[pl.ds(i0 + PAD - d, BL), :]
        else:
            kw = kp_ref[pl.ds(i0 + PAD - 8, BL + 8), :]
            vw = vp_ref[pl.ds(i0 + PAD - 8, BL + 8), :]
            ks = kw[8 - d:8 - d + BL, :]
            vs = vw[8 - d:8 - d + BL, :]
        s = jnp.dot(q * ks, sel, preferred_element_type=f32) * SCALE
        w = jnp.exp(s) - 1.0
        z += w
        acc += jnp.dot(w, selT, preferred_element_type=f32) * vs

    att = (acc + sv_ref[...]) / jnp.dot(z, selT, preferred_element_type=f32)
    o_ref[...] = jnp.dot(att, wo_ref[...], preferred_element_type=f32) + bo_ref[...]


@jax.jit
def kernel(x, Wq, bq, Wk, bk, Wv, bv, Wo, bo):
    x2 = x.reshape(L, D)
    bq2 = bq.reshape(1, D)
    bk2 = bk.reshape(1, D)
    bv2 = bv.reshape(1, D)
    bo2 = bo.reshape(1, D)
    nblk = L // BL

    full = lambda shape: pl.BlockSpec(shape, lambda i: (0, 0))
    rows = pl.BlockSpec((BL, D), lambda i: (i, 0))

    q, k, v, sv = pl.pallas_call(
        _proj_kernel,
        grid=(nblk,),
        in_specs=[rows, full((D, D)), full((D, D)), full((D, D)),
                  full((1, D)), full((1, D)), full((1, D))],
        out_specs=[rows, rows, rows, full((1, D))],
        out_shape=[
            jax.ShapeDtypeStruct((L, D), jnp.float32),
            jax.ShapeDtypeStruct((L, D), jnp.float32),
            jax.ShapeDtypeStruct((L, D), jnp.float32),
            jax.ShapeDtypeStruct((1, D), jnp.float32),
        ],
        compiler_params=pltpu.CompilerParams(
            dimension_semantics=("arbitrary",)),
    )(x2, Wq, Wk, Wv, bq2, bk2, bv2)

    zpad = jnp.zeros((PAD, D), jnp.float32)
    kp = jnp.concatenate([zpad, k], axis=0)
    vp = jnp.concatenate([zpad, v], axis=0)

    out = pl.pallas_call(
        _attn_kernel,
        grid=(nblk,),
        in_specs=[rows, full((PAD + L, D)), full((PAD + L, D)),
                  full((1, D)), full((D, D)), full((1, D))],
        out_specs=rows,
        out_shape=jax.ShapeDtypeStruct((L, D), jnp.float32),
        compiler_params=pltpu.CompilerParams(
            dimension_semantics=("arbitrary",)),
    )(q, kp, vp, sv, Wo, bo2)

    return out.reshape(1, L, D)


# speedup vs baseline: 4.5510x; 1.4724x over previous
import math

import jax
import jax.numpy as jnp
from jax import lax
from jax.experimental import pallas as pl
from jax.experimental.pallas import tpu as pltpu

L = 2048
D = 1024
H = 16
DH = 64
BL = 256
NBLK = L // BL
PAD = 1024
OFFSETS = tuple(2 ** j for j in range(11))
SCALE = 1.0 / math.sqrt(DH)


def _fused_kernel(x_ref, wq_ref, wk_ref, wv_ref, wo_ref,
                  bq_ref, bk_ref, bv_ref, bo_ref, o_ref,
                  q_s, kp_s, vp_s, sv_s):
    f32 = jnp.float32
    pid = pl.program_id(0)

    @pl.when(pid < NBLK)
    def _proj():
        r0 = pid * BL
        xb = x_ref[...]
        q_s[pl.ds(r0, BL), :] = (
            jnp.dot(xb, wq_ref[...], preferred_element_type=f32) + bq_ref[...])
        kp_s[pl.ds(PAD + r0, BL), :] = (
            jnp.dot(xb, wk_ref[...], preferred_element_type=f32) + bk_ref[...])
        vb = jnp.dot(xb, wv_ref[...], preferred_element_type=f32) + bv_ref[...]
        vp_s[pl.ds(PAD + r0, BL), :] = vb

        @pl.when(pid < PAD // BL)
        def _():
            kp_s[pl.ds(pid * BL, BL), :] = jnp.zeros((BL, D), f32)
            vp_s[pl.ds(pid * BL, BL), :] = jnp.zeros((BL, D), f32)

        sv = jnp.sum(vb, axis=0, keepdims=True)

        @pl.when(pid == 0)
        def _():
            sv_s[...] = sv

        @pl.when(pid != 0)
        def _():
            sv_s[...] += sv

    @pl.when(pid >= NBLK)
    def _attn():
        i0 = (pid - NBLK) * BL
        q = q_s[pl.ds(i0, BL), :]

        sel = (lax.broadcasted_iota(jnp.int32, (D, H), 0) // DH
               == lax.broadcasted_iota(jnp.int32, (D, H), 1)).astype(f32)
        selT = (lax.broadcasted_iota(jnp.int32, (H, D), 1) // DH
                == lax.broadcasted_iota(jnp.int32, (H, D), 0)).astype(f32)

        kd = kp_s[pl.ds(i0 + PAD, BL), :]
        vd = vp_s[pl.ds(i0 + PAD, BL), :]
        s = jnp.dot(q * kd, sel, preferred_element_type=f32) * SCALE
        w = jnp.exp(s) - 1.0
        z = w + float(L)
        acc = jnp.dot(w, selT, preferred_element_type=f32) * vd

        for d in OFFSETS:
            if d % 8 == 0:
                ks = kp_s[pl.ds(i0 + PAD - d, BL), :]
                vs = vp_s[pl.ds(i0 + PAD - d, BL), :]
            else:
                kw = kp_s[pl.ds(i0 + PAD - 8, BL + 8), :]
                vw = vp_s[pl.ds(i0 + PAD - 8, BL + 8), :]
                ks = kw[8 - d:8 - d + BL, :]
                vs = vw[8 - d:8 - d + BL, :]
            s = jnp.dot(q * ks, sel, preferred_element_type=f32) * SCALE
            w = jnp.exp(s) - 1.0
            z += w
            acc += jnp.dot(w, selT, preferred_element_type=f32) * vs

        att = (acc + sv_s[...]) / jnp.dot(z, selT, preferred_element_type=f32)
        o_ref[...] = (
            jnp.dot(att, wo_ref[...], preferred_element_type=f32) + bo_ref[...])


@jax.jit
def kernel(x, Wq, bq, Wk, bk, Wv, bv, Wo, bo):
    x2 = x.reshape(L, D)
    bq2 = bq.reshape(1, D)
    bk2 = bk.reshape(1, D)
    bv2 = bv.reshape(1, D)
    bo2 = bo.reshape(1, D)

    full = lambda shape: pl.BlockSpec(shape, lambda i: (0, 0))

    out = pl.pallas_call(
        _fused_kernel,
        grid=(2 * NBLK,),
        in_specs=[
            pl.BlockSpec((BL, D), lambda i: (jnp.minimum(i, NBLK - 1), 0)),
            full((D, D)), full((D, D)), full((D, D)), full((D, D)),
            full((1, D)), full((1, D)), full((1, D)), full((1, D)),
        ],
        out_specs=pl.BlockSpec((BL, D), lambda i: (jnp.maximum(i - NBLK, 0), 0)),
        out_shape=jax.ShapeDtypeStruct((L, D), jnp.float32),
        scratch_shapes=[
            pltpu.VMEM((L, D), jnp.float32),
            pltpu.VMEM((PAD + L, D), jnp.float32),
            pltpu.VMEM((PAD + L, D), jnp.float32),
            pltpu.VMEM((1, D), jnp.float32),
        ],
        compiler_params=pltpu.CompilerParams(
            dimension_semantics=("arbitrary",)),
    )(x2, Wq, Wk, Wv, Wo, bq2, bk2, bv2, bo2)

    return out.reshape(1, L, D)
